# Initial kernel scaffold; baseline (speedup 1.0000x reference)
import jax
import jax.numpy as jnp
from jax.experimental import pallas as pl


def kernel(node_feats, edge_feats, edge_index, W1, b1, W2, b2, gamma, beta):
    raise NotImplementedError("write your pallas kernel here")


# speedup vs baseline: 3.1277x; 3.1277x over previous
import functools
import math

import jax
import jax.numpy as jnp
from jax import lax
from jax.experimental import pallas as pl
from jax.experimental.pallas import tpu as pltpu
from jax.experimental.pallas import tpu_sc as plsc

CHUNK = 128


def _make_sc_segment_sum(N, E, D):
    NC, NS = 2, 16
    DC = D // NC
    NCH = E // CHUNK
    assert NCH * CHUNK == E and DC * NC == D
    ZR = 80
    NB = N // ZR
    assert NB * ZR == N
    qb, rb = divmod(NB, NS)
    q, r = divmod(NCH, NS)

    mesh = plsc.VectorSubcoreMesh(core_axis_name="c", subcore_axis_name="s")

    @functools.partial(
        pl.kernel,
        out_type=jax.ShapeDtypeStruct((N, D), jnp.float32),
        mesh=mesh,
        scratch_types=[
            pltpu.VMEM((CHUNK,), jnp.int32),
            pltpu.VMEM((CHUNK,), jnp.int32),
            pltpu.VMEM((CHUNK, DC), jnp.float32),
            pltpu.VMEM((CHUNK, DC), jnp.float32),
            pltpu.VMEM((ZR, DC), jnp.float32),
            pltpu.VMEM_SHARED((N, DC), jnp.float32),
            pltpu.SemaphoreType.DMA,
        ],
    )
    def sc_seg_sum(node_lo, node_hi, edge_feats, src1, dst1, out,
                   src_v, dst_v, ebuf, nbuf, zbuf, agg_sh, sem):
        cid = lax.axis_index("c")
        sid = lax.axis_index("s")
        col0 = pl.multiple_of(cid * DC, DC)

        zvec = jnp.zeros((16,), jnp.float32)

        def zero_body(i, _):
            rr = i // (DC // 16)
            cc = i % (DC // 16)
            zbuf[rr, pl.ds(cc * 16, 16)] = zvec
            return 0

        lax.fori_loop(0, ZR * (DC // 16), zero_body, 0)
        blk0 = sid * qb + jnp.minimum(sid, rb)
        nblk = jnp.where(sid < rb, qb + 1, qb)

        def zcopy_body(k, _):
            r0 = pl.multiple_of((blk0 + k) * ZR, 8)
            pltpu.sync_copy(zbuf, agg_sh.at[pl.ds(r0, ZR)])
            return 0

        lax.fori_loop(0, nblk, zcopy_body, 0)
        plsc.subcore_barrier()

        base = sid * q + jnp.minimum(sid, r)
        nch = jnp.where(sid < r, q + 1, q)

        def body(j, _):
            g = base + j
            e0 = pl.multiple_of(g * CHUNK, CHUNK)
            pltpu.sync_copy(src1.at[pl.ds(e0, CHUNK)], src_v)
            pltpu.sync_copy(dst1.at[pl.ds(e0, CHUNK)], dst_v)
            pltpu.sync_copy(edge_feats.at[pl.ds(e0, CHUNK), pl.ds(col0, DC)],
                            ebuf)

            @pl.when(cid == 0)
            def _():
                pltpu.async_copy(node_lo.at[src_v], nbuf, sem).wait()

            @pl.when(cid == 1)
            def _():
                pltpu.async_copy(node_hi.at[src_v], nbuf, sem).wait()

            pltpu.sync_copy(ebuf, agg_sh.at[dst_v], add=True)
            pltpu.sync_copy(nbuf, agg_sh.at[dst_v], add=True)
            return 0

        lax.fori_loop(0, nch, body, 0)
        plsc.subcore_barrier()

        def wb_body(k, _):
            r0 = pl.multiple_of((blk0 + k) * ZR, 8)
            pltpu.sync_copy(agg_sh.at[pl.ds(r0, ZR)],
                            out.at[pl.ds(r0, ZR), pl.ds(col0, DC)])
            return 0

        lax.fori_loop(0, nblk, wb_body, 0)

    return sc_seg_sum


def _tc_mlp_ln(agg, node_feats, W1, b1, W2, b2, gamma, beta):
    N, D = node_feats.shape
    H = W1.shape[1]
    BR = 1000
    inv_sqrt_n = 1.0 / math.sqrt(float(N))

    def body(agg_ref, node_ref, w1_ref, b1_ref, w2_ref, b2_ref, g_ref, be_ref,
             out_ref):
        a = agg_ref[...]
        h = jnp.dot(a, w1_ref[...], preferred_element_type=jnp.float32)
        h = jnp.maximum(h + b1_ref[...], 0.0)
        h = jnp.dot(h, w2_ref[...], preferred_element_type=jnp.float32)
        h = h + b2_ref[...]
        mean = jnp.mean(h, axis=-1, keepdims=True)
        c = h - mean
        var = jnp.mean(c * c, axis=-1, keepdims=True)
        h = c * lax.rsqrt(var + 1e-5) * g_ref[...] + be_ref[...]
        h = jnp.maximum(h * inv_sqrt_n, 0.0)
        out_ref[...] = h + node_ref[...]

    grid = (N // BR,)
    return pl.pallas_call(
        body,
        grid=grid,
        in_specs=[
            pl.BlockSpec((BR, D), lambda i: (i, 0)),
            pl.BlockSpec((BR, D), lambda i: (i, 0)),
            pl.BlockSpec((D, H), lambda i: (0, 0)),
            pl.BlockSpec((1, H), lambda i: (0, 0)),
            pl.BlockSpec((H, D), lambda i: (0, 0)),
            pl.BlockSpec((1, D), lambda i: (0, 0)),
            pl.BlockSpec((1, D), lambda i: (0, 0)),
            pl.BlockSpec((1, D), lambda i: (0, 0)),
        ],
        out_specs=pl.BlockSpec((BR, D), lambda i: (i, 0)),
        out_shape=jax.ShapeDtypeStruct((N, D), jnp.float32),
    )(agg, node_feats, W1, b1.reshape(1, H), W2, b2.reshape(1, D),
      gamma.reshape(1, D), beta.reshape(1, D))


def kernel(node_feats, edge_feats, edge_index, W1, b1, W2, b2, gamma, beta):
    N, D = node_feats.shape
    E = edge_feats.shape[0]
    DC = D // 2

    src1 = edge_index[0]
    dst1 = edge_index[1]
    node_lo = node_feats[:, :DC]
    node_hi = node_feats[:, DC:]

    sc_seg_sum = _make_sc_segment_sum(N, E, D)
    agg = sc_seg_sum(node_lo, node_hi, edge_feats, src1, dst1)

    return _tc_mlp_ln(agg, node_feats, W1, b1, W2, b2, gamma, beta)


# speedup vs baseline: 4.5770x; 1.4634x over previous
import functools
import math

import jax
import jax.numpy as jnp
from jax import lax
from jax.experimental import pallas as pl
from jax.experimental.pallas import tpu as pltpu
from jax.experimental.pallas import tpu_sc as plsc

CHUNK = 80


def _make_sc_segment_sum(N, E, D):
    NC, NS = 2, 16
    DC = D // NC
    NCH = E // CHUNK
    assert NCH * CHUNK == E and DC * NC == D
    assert CHUNK % 8 == 0
    ZR = 40
    NB = N // ZR
    assert NB * ZR == N
    qb, rb = divmod(NB, NS)
    qp, rp = divmod(NCH // 2, NS)
    assert NCH % 2 == 0 and qp > 0

    mesh = plsc.VectorSubcoreMesh(core_axis_name="c", subcore_axis_name="s")

    @functools.partial(
        pl.kernel,
        out_type=jax.ShapeDtypeStruct((N, D), jnp.float32),
        mesh=mesh,
        scratch_types=[
            pltpu.VMEM((CHUNK,), jnp.int32),
            pltpu.VMEM((CHUNK,), jnp.int32),
            pltpu.VMEM((CHUNK,), jnp.int32),
            pltpu.VMEM((CHUNK,), jnp.int32),
            pltpu.VMEM((CHUNK, DC), jnp.float32),
            pltpu.VMEM((CHUNK, DC), jnp.float32),
            pltpu.VMEM((CHUNK, DC), jnp.float32),
            pltpu.VMEM((CHUNK, DC), jnp.float32),
            pltpu.VMEM_SHARED((N, DC), jnp.float32),
            pltpu.SemaphoreType.DMA,
            pltpu.SemaphoreType.DMA,
        ],
    )
    def sc_seg_sum(node_lo, node_hi, edge_feats, src1, dst1, out,
                   src_i0, src_i1, dst_i0, dst_i1, ebuf0, ebuf1, nbuf0, nbuf1,
                   agg_sh, sem_e, sem_n):
        src_i = (src_i0, src_i1)
        dst_i = (dst_i0, dst_i1)
        ebuf = (ebuf0, ebuf1)
        nbuf = (nbuf0, nbuf1)
        cid = lax.axis_index("c")
        sid = lax.axis_index("s")
        col0 = pl.multiple_of(cid * DC, DC)

        zvec = jnp.zeros((16,), jnp.float32)

        def zero_body(i, _):
            rr = i // (DC // 16)
            cc = i % (DC // 16)
            ebuf0[rr, pl.ds(cc * 16, 16)] = zvec
            return 0

        lax.fori_loop(0, ZR * (DC // 16), zero_body, 0)
        blk0 = sid * qb + jnp.minimum(sid, rb)
        nblk = jnp.where(sid < rb, qb + 1, qb)

        def zcopy_body(k, _):
            r0 = pl.multiple_of((blk0 + k) * ZR, 8)
            pltpu.sync_copy(ebuf0.at[pl.ds(0, ZR)], agg_sh.at[pl.ds(r0, ZR)])
            return 0

        lax.fori_loop(0, nblk, zcopy_body, 0)
        plsc.subcore_barrier()

        base = (sid * qp + jnp.minimum(sid, rp)) * 2
        nch = jnp.where(sid < rp, qp + 1, qp) * 2

        def issue_idx(j, p):
            e0 = pl.multiple_of((base + j) * CHUNK, 8)
            pltpu.sync_copy(src1.at[pl.ds(e0, CHUNK)], src_i[p])
            pltpu.sync_copy(dst1.at[pl.ds(e0, CHUNK)], dst_i[p])

        def issue_edge(j, p):
            e0 = pl.multiple_of((base + j) * CHUNK, 8)
            pltpu.async_copy(edge_feats.at[pl.ds(e0, CHUNK), pl.ds(col0, DC)],
                            ebuf[p], sem_e)

        def wait_edge(p):
            pltpu.make_async_copy(
                edge_feats.at[pl.ds(0, CHUNK), pl.ds(0, DC)],
                ebuf[p], sem_e).wait()

        def issue_gather(p):
            @pl.when(cid == 0)
            def _():
                pltpu.async_copy(node_lo.at[src_i[p]], nbuf[p], sem_n)

            @pl.when(cid == 1)
            def _():
                pltpu.async_copy(node_hi.at[src_i[p]], nbuf[p], sem_n)

        def wait_gather(p):
            pltpu.make_async_copy(node_lo.at[pl.ds(0, CHUNK)], nbuf[p],
                                  sem_n).wait()

        issue_idx(0, 0)
        issue_idx(1, 1)
        issue_edge(0, 0)
        issue_edge(1, 1)
        issue_gather(0)

        def body(jj, _):
            for p in (0, 1):
                j = 2 * jj + p
                q = 1 - p

                @pl.when(j + 1 < nch)
                def _():
                    issue_gather(q)

                wait_edge(p)
                wait_gather(p)
                pltpu.sync_copy(ebuf[p], agg_sh.at[dst_i[p]], add=True)
                pltpu.sync_copy(nbuf[p], agg_sh.at[dst_i[p]], add=True)

                @pl.when(j + 2 < nch)
                def _():
                    issue_idx(j + 2, p)
                    issue_edge(j + 2, p)
            return 0

        lax.fori_loop(0, nch // 2, body, 0)
        plsc.subcore_barrier()

        def wb_body(k, _):
            r0 = pl.multiple_of((blk0 + k) * ZR, 8)
            pltpu.sync_copy(agg_sh.at[pl.ds(r0, ZR)],
                            out.at[pl.ds(r0, ZR), pl.ds(col0, DC)])
            return 0

        lax.fori_loop(0, nblk, wb_body, 0)

    return sc_seg_sum


def _tc_mlp_ln(agg, node_feats, W1, b1, W2, b2, gamma, beta):
    N, D = node_feats.shape
    H = W1.shape[1]
    BR = 1000
    inv_sqrt_n = 1.0 / math.sqrt(float(N))

    def body(agg_ref, node_ref, w1_ref, b1_ref, w2_ref, b2_ref, g_ref, be_ref,
             out_ref):
        a = agg_ref[...]
        h = jnp.dot(a, w1_ref[...], preferred_element_type=jnp.float32)
        h = jnp.maximum(h + b1_ref[...], 0.0)
        h = jnp.dot(h, w2_ref[...], preferred_element_type=jnp.float32)
        h = h + b2_ref[...]
        mean = jnp.mean(h, axis=-1, keepdims=True)
        c = h - mean
        var = jnp.mean(c * c, axis=-1, keepdims=True)
        h = c * lax.rsqrt(var + 1e-5) * g_ref[...] + be_ref[...]
        h = jnp.maximum(h * inv_sqrt_n, 0.0)
        out_ref[...] = h + node_ref[...]

    grid = (N // BR,)
    return pl.pallas_call(
        body,
        grid=grid,
        in_specs=[
            pl.BlockSpec((BR, D), lambda i: (i, 0)),
            pl.BlockSpec((BR, D), lambda i: (i, 0)),
            pl.BlockSpec((D, H), lambda i: (0, 0)),
            pl.BlockSpec((1, H), lambda i: (0, 0)),
            pl.BlockSpec((H, D), lambda i: (0, 0)),
            pl.BlockSpec((1, D), lambda i: (0, 0)),
            pl.BlockSpec((1, D), lambda i: (0, 0)),
            pl.BlockSpec((1, D), lambda i: (0, 0)),
        ],
        out_specs=pl.BlockSpec((BR, D), lambda i: (i, 0)),
        out_shape=jax.ShapeDtypeStruct((N, D), jnp.float32),
    )(agg, node_feats, W1, b1.reshape(1, H), W2, b2.reshape(1, D),
      gamma.reshape(1, D), beta.reshape(1, D))


def kernel(node_feats, edge_feats, edge_index, W1, b1, W2, b2, gamma, beta):
    N, D = node_feats.shape
    E = edge_feats.shape[0]
    DC = D // 2

    src1 = edge_index[0]
    dst1 = edge_index[1]
    node_lo = node_feats[:, :DC]
    node_hi = node_feats[:, DC:]

    sc_seg_sum = _make_sc_segment_sum(N, E, D)
    agg = sc_seg_sum(node_lo, node_hi, edge_feats, src1, dst1)

    return _tc_mlp_ln(agg, node_feats, W1, b1, W2, b2, gamma, beta)


# speedup vs baseline: 5.6895x; 1.2431x over previous
import functools
import math

import jax
import jax.numpy as jnp
from jax import lax
from jax.experimental import pallas as pl
from jax.experimental.pallas import tpu as pltpu
from jax.experimental.pallas import tpu_sc as plsc

CHUNK = 80


def _make_sc_segment_sum(N, E, D):
    NC, NS = 2, 16
    DC = D // NC
    NCH = E // CHUNK
    assert NCH * CHUNK == E and DC * NC == D
    assert CHUNK % 8 == 0
    ZR = 40
    NB = N // ZR
    assert NB * ZR == N
    qb, rb = divmod(NB, NS)
    qp, rp = divmod(NCH // 2, NS)
    assert NCH % 2 == 0 and qp > 0

    mesh = plsc.VectorSubcoreMesh(core_axis_name="c", subcore_axis_name="s")

    @functools.partial(
        pl.kernel,
        out_type=jax.ShapeDtypeStruct((N, D), jnp.float32),
        mesh=mesh,
        scratch_types=[
            pltpu.VMEM((CHUNK,), jnp.int32),
            pltpu.VMEM((CHUNK,), jnp.int32),
            pltpu.VMEM((CHUNK,), jnp.int32),
            pltpu.VMEM((CHUNK,), jnp.int32),
            pltpu.VMEM((CHUNK, DC), jnp.float32),
            pltpu.VMEM((CHUNK, DC), jnp.float32),
            pltpu.VMEM((CHUNK, DC), jnp.float32),
            pltpu.VMEM((CHUNK, DC), jnp.float32),
            pltpu.VMEM_SHARED((N, DC), jnp.float32),
            pltpu.SemaphoreType.DMA,
            pltpu.SemaphoreType.DMA,
            pltpu.SemaphoreType.DMA,
        ],
    )
    def sc_seg_sum(node_lo, node_hi, edge_feats, src1, dst1, out,
                   src_i0, src_i1, dst_i0, dst_i1, ebuf0, ebuf1, nbuf0, nbuf1,
                   agg_sh, sem_i, sem_e, sem_n):
        src_i = (src_i0, src_i1)
        dst_i = (dst_i0, dst_i1)
        ebuf = (ebuf0, ebuf1)
        nbuf = (nbuf0, nbuf1)
        cid = lax.axis_index("c")
        sid = lax.axis_index("s")
        col0 = pl.multiple_of(cid * DC, DC)

        zvec = jnp.zeros((16,), jnp.float32)

        def zero_body(i, _):
            rr = i // (DC // 16)
            cc = i % (DC // 16)
            ebuf0[rr, pl.ds(cc * 16, 16)] = zvec
            return 0

        lax.fori_loop(0, ZR * (DC // 16), zero_body, 0)
        blk0 = sid * qb + jnp.minimum(sid, rb)
        nblk = jnp.where(sid < rb, qb + 1, qb)

        def zcopy_body(k, _):
            r0 = pl.multiple_of((blk0 + k) * ZR, 8)
            pltpu.sync_copy(ebuf0.at[pl.ds(0, ZR)], agg_sh.at[pl.ds(r0, ZR)])
            return 0

        lax.fori_loop(0, nblk, zcopy_body, 0)
        plsc.subcore_barrier()

        base = (sid * qp + jnp.minimum(sid, rp)) * 2
        nch = jnp.where(sid < rp, qp + 1, qp) * 2

        def issue_idx(j, p):
            e0 = pl.multiple_of((base + j) * CHUNK, 8)
            pltpu.async_copy(src1.at[pl.ds(e0, CHUNK)], src_i[p], sem_i)
            pltpu.async_copy(dst1.at[pl.ds(e0, CHUNK)], dst_i[p], sem_i)

        def wait_idx(p):
            pltpu.make_async_copy(src1.at[pl.ds(0, CHUNK)], src_i[p],
                                  sem_i).wait()
            pltpu.make_async_copy(dst1.at[pl.ds(0, CHUNK)], dst_i[p],
                                  sem_i).wait()

        def issue_edge(j, p):
            e0 = pl.multiple_of((base + j) * CHUNK, 8)
            pltpu.async_copy(edge_feats.at[pl.ds(e0, CHUNK), pl.ds(col0, DC)],
                            ebuf[p], sem_e)

        def wait_edge(p):
            pltpu.make_async_copy(
                edge_feats.at[pl.ds(0, CHUNK), pl.ds(0, DC)],
                ebuf[p], sem_e).wait()

        def issue_gather(p):
            @pl.when(cid == 0)
            def _():
                pltpu.async_copy(node_lo.at[src_i[p]], nbuf[p], sem_n)

            @pl.when(cid == 1)
            def _():
                pltpu.async_copy(node_hi.at[src_i[p]], nbuf[p], sem_n)

        def wait_gather(p):
            pltpu.make_async_copy(node_lo.at[pl.ds(0, CHUNK)], nbuf[p],
                                  sem_n).wait()

        issue_idx(0, 0)
        issue_idx(1, 1)
        issue_edge(0, 0)
        issue_edge(1, 1)
        wait_idx(0)
        issue_gather(0)

        def body(jj, _):
            for p in (0, 1):
                j = 2 * jj + p
                q = 1 - p

                @pl.when(j + 1 < nch)
                def _():
                    wait_idx(q)
                    issue_gather(q)

                wait_edge(p)
                wait_gather(p)
                pltpu.sync_copy(ebuf[p], agg_sh.at[dst_i[p]], add=True)
                pltpu.sync_copy(nbuf[p], agg_sh.at[dst_i[p]], add=True)

                @pl.when(j + 2 < nch)
                def _():
                    issue_idx(j + 2, p)
                    issue_edge(j + 2, p)
            return 0

        lax.fori_loop(0, nch // 2, body, 0)
        plsc.subcore_barrier()

        def wb_body(k, _):
            r0 = pl.multiple_of((blk0 + k) * ZR, 8)
            pltpu.sync_copy(agg_sh.at[pl.ds(r0, ZR)],
                            out.at[pl.ds(r0, ZR), pl.ds(col0, DC)])
            return 0

        lax.fori_loop(0, nblk, wb_body, 0)

    return sc_seg_sum


def _tc_mlp_ln(agg, node_feats, W1, b1, W2, b2, gamma, beta):
    N, D = node_feats.shape
    H = W1.shape[1]
    BR = 1000
    inv_sqrt_n = 1.0 / math.sqrt(float(N))

    def body(agg_ref, node_ref, w1_ref, b1_ref, w2_ref, b2_ref, g_ref, be_ref,
             out_ref):
        a = agg_ref[...]
        h = jnp.dot(a, w1_ref[...], preferred_element_type=jnp.float32)
        h = jnp.maximum(h + b1_ref[...], 0.0)
        h = jnp.dot(h, w2_ref[...], preferred_element_type=jnp.float32)
        h = h + b2_ref[...]
        mean = jnp.mean(h, axis=-1, keepdims=True)
        c = h - mean
        var = jnp.mean(c * c, axis=-1, keepdims=True)
        h = c * lax.rsqrt(var + 1e-5) * g_ref[...] + be_ref[...]
        h = jnp.maximum(h * inv_sqrt_n, 0.0)
        out_ref[...] = h + node_ref[...]

    grid = (N // BR,)
    return pl.pallas_call(
        body,
        grid=grid,
        in_specs=[
            pl.BlockSpec((BR, D), lambda i: (i, 0)),
            pl.BlockSpec((BR, D), lambda i: (i, 0)),
            pl.BlockSpec((D, H), lambda i: (0, 0)),
            pl.BlockSpec((1, H), lambda i: (0, 0)),
            pl.BlockSpec((H, D), lambda i: (0, 0)),
            pl.BlockSpec((1, D), lambda i: (0, 0)),
            pl.BlockSpec((1, D), lambda i: (0, 0)),
            pl.BlockSpec((1, D), lambda i: (0, 0)),
        ],
        out_specs=pl.BlockSpec((BR, D), lambda i: (i, 0)),
        out_shape=jax.ShapeDtypeStruct((N, D), jnp.float32),
    )(agg, node_feats, W1, b1.reshape(1, H), W2, b2.reshape(1, D),
      gamma.reshape(1, D), beta.reshape(1, D))


def kernel(node_feats, edge_feats, edge_index, W1, b1, W2, b2, gamma, beta):
    N, D = node_feats.shape
    E = edge_feats.shape[0]
    DC = D // 2

    src1 = edge_index[0]
    dst1 = edge_index[1]
    node_lo = node_feats[:, :DC]
    node_hi = node_feats[:, DC:]

    sc_seg_sum = _make_sc_segment_sum(N, E, D)
    agg = sc_seg_sum(node_lo, node_hi, edge_feats, src1, dst1)

    return _tc_mlp_ln(agg, node_feats, W1, b1, W2, b2, gamma, beta)


# speedup vs baseline: 5.8302x; 1.0247x over previous
import functools
import math

import jax
import jax.numpy as jnp
from jax import lax
from jax.experimental import pallas as pl
from jax.experimental.pallas import tpu as pltpu
from jax.experimental.pallas import tpu_sc as plsc

CHUNK = 80


def _make_sc_segment_sum(N, E, D):
    NC, NS = 2, 16
    DC = D // NC
    NCH = E // CHUNK
    assert NCH * CHUNK == E and DC * NC == D
    assert CHUNK % 8 == 0
    ZR = 40
    NB = N // ZR
    assert NB * ZR == N
    qb, rb = divmod(NB, NS)
    qp, rp = divmod(NCH // 2, NS)
    assert NCH % 2 == 0 and qp > 0

    mesh = plsc.VectorSubcoreMesh(core_axis_name="c", subcore_axis_name="s")

    @functools.partial(
        pl.kernel,
        out_type=jax.ShapeDtypeStruct((N, D), jnp.float32),
        mesh=mesh,
        scratch_types=[
            pltpu.VMEM((CHUNK,), jnp.int32),
            pltpu.VMEM((CHUNK,), jnp.int32),
            pltpu.VMEM((CHUNK,), jnp.int32),
            pltpu.VMEM((CHUNK,), jnp.int32),
            pltpu.VMEM((CHUNK, DC), jnp.float32),
            pltpu.VMEM((CHUNK, DC), jnp.float32),
            pltpu.VMEM((CHUNK, DC), jnp.float32),
            pltpu.VMEM((CHUNK, DC), jnp.float32),
            pltpu.VMEM_SHARED((N, DC), jnp.float32),
            pltpu.SemaphoreType.DMA,
            pltpu.SemaphoreType.DMA,
            pltpu.SemaphoreType.DMA,
            pltpu.SemaphoreType.DMA,
        ],
    )
    def sc_seg_sum(node_lo, node_hi, edge_feats, src1, dst1, out,
                   src_i0, src_i1, dst_i0, dst_i1, ebuf0, ebuf1, nbuf0, nbuf1,
                   agg_sh, sem_i, sem_e, sem_n, sem_s):
        src_i = (src_i0, src_i1)
        dst_i = (dst_i0, dst_i1)
        ebuf = (ebuf0, ebuf1)
        nbuf = (nbuf0, nbuf1)
        cid = lax.axis_index("c")
        sid = lax.axis_index("s")
        col0 = pl.multiple_of(cid * DC, DC)

        zvec = jnp.zeros((16,), jnp.float32)

        def zero_body(i, _):
            rr = i // (DC // 16)
            cc = i % (DC // 16)
            ebuf0[rr, pl.ds(cc * 16, 16)] = zvec
            return 0

        lax.fori_loop(0, ZR * (DC // 16), zero_body, 0)
        blk0 = sid * qb + jnp.minimum(sid, rb)
        nblk = jnp.where(sid < rb, qb + 1, qb)

        def zcopy_body(k, _):
            r0 = pl.multiple_of((blk0 + k) * ZR, 8)
            pltpu.sync_copy(ebuf0.at[pl.ds(0, ZR)], agg_sh.at[pl.ds(r0, ZR)])
            return 0

        lax.fori_loop(0, nblk, zcopy_body, 0)
        plsc.subcore_barrier()

        base = (sid * qp + jnp.minimum(sid, rp)) * 2
        nch = jnp.where(sid < rp, qp + 1, qp) * 2

        def issue_idx(j, p):
            e0 = pl.multiple_of((base + j) * CHUNK, 8)
            pltpu.async_copy(src1.at[pl.ds(e0, CHUNK)], src_i[p], sem_i)
            pltpu.async_copy(dst1.at[pl.ds(e0, CHUNK)], dst_i[p], sem_i)

        def wait_idx(p):
            pltpu.make_async_copy(src1.at[pl.ds(0, CHUNK)], src_i[p],
                                  sem_i).wait()
            pltpu.make_async_copy(dst1.at[pl.ds(0, CHUNK)], dst_i[p],
                                  sem_i).wait()

        def issue_edge(j, p):
            e0 = pl.multiple_of((base + j) * CHUNK, 8)
            pltpu.async_copy(edge_feats.at[pl.ds(e0, CHUNK), pl.ds(col0, DC)],
                            ebuf[p], sem_e)

        def wait_edge(p):
            pltpu.make_async_copy(
                edge_feats.at[pl.ds(0, CHUNK), pl.ds(0, DC)],
                ebuf[p], sem_e).wait()

        def issue_gather(p):
            @pl.when(cid == 0)
            def _():
                pltpu.async_copy(node_lo.at[src_i[p]], nbuf[p], sem_n)

            @pl.when(cid == 1)
            def _():
                pltpu.async_copy(node_hi.at[src_i[p]], nbuf[p], sem_n)

        def wait_gather(p):
            pltpu.make_async_copy(node_lo.at[pl.ds(0, CHUNK)], nbuf[p],
                                  sem_n).wait()

        issue_idx(0, 0)
        issue_idx(1, 1)
        issue_edge(0, 0)
        issue_edge(1, 1)
        wait_idx(0)
        issue_gather(0)

        def body(jj, _):
            for p in (0, 1):
                j = 2 * jj + p
                q = 1 - p

                @pl.when(j + 1 < nch)
                def _():
                    wait_idx(q)
                    issue_gather(q)

                wait_edge(p)
                wait_gather(p)
                pltpu.async_copy(ebuf[p], agg_sh.at[dst_i[p]], sem_s,
                                 add=True)
                pltpu.sync_copy(nbuf[p], agg_sh.at[dst_i[p]], add=True)
                pltpu.make_async_copy(ebuf[p], agg_sh.at[pl.ds(0, CHUNK)],
                                      sem_s).wait()

                @pl.when(j + 2 < nch)
                def _():
                    issue_idx(j + 2, p)
                    issue_edge(j + 2, p)
            return 0

        lax.fori_loop(0, nch // 2, body, 0)
        plsc.subcore_barrier()

        def wb_body(k, _):
            r0 = pl.multiple_of((blk0 + k) * ZR, 8)
            pltpu.sync_copy(agg_sh.at[pl.ds(r0, ZR)],
                            out.at[pl.ds(r0, ZR), pl.ds(col0, DC)])
            return 0

        lax.fori_loop(0, nblk, wb_body, 0)

    return sc_seg_sum


def _tc_mlp_ln(agg, node_feats, W1, b1, W2, b2, gamma, beta):
    N, D = node_feats.shape
    H = W1.shape[1]
    BR = 1000
    inv_sqrt_n = 1.0 / math.sqrt(float(N))

    def body(agg_ref, node_ref, w1_ref, b1_ref, w2_ref, b2_ref, g_ref, be_ref,
             out_ref):
        a = agg_ref[...]
        h = jnp.dot(a, w1_ref[...], preferred_element_type=jnp.float32)
        h = jnp.maximum(h + b1_ref[...], 0.0)
        h = jnp.dot(h, w2_ref[...], preferred_element_type=jnp.float32)
        h = h + b2_ref[...]
        mean = jnp.mean(h, axis=-1, keepdims=True)
        c = h - mean
        var = jnp.mean(c * c, axis=-1, keepdims=True)
        h = c * lax.rsqrt(var + 1e-5) * g_ref[...] + be_ref[...]
        h = jnp.maximum(h * inv_sqrt_n, 0.0)
        out_ref[...] = h + node_ref[...]

    grid = (N // BR,)
    return pl.pallas_call(
        body,
        grid=grid,
        in_specs=[
            pl.BlockSpec((BR, D), lambda i: (i, 0)),
            pl.BlockSpec((BR, D), lambda i: (i, 0)),
            pl.BlockSpec((D, H), lambda i: (0, 0)),
            pl.BlockSpec((1, H), lambda i: (0, 0)),
            pl.BlockSpec((H, D), lambda i: (0, 0)),
            pl.BlockSpec((1, D), lambda i: (0, 0)),
            pl.BlockSpec((1, D), lambda i: (0, 0)),
            pl.BlockSpec((1, D), lambda i: (0, 0)),
        ],
        out_specs=pl.BlockSpec((BR, D), lambda i: (i, 0)),
        out_shape=jax.ShapeDtypeStruct((N, D), jnp.float32),
    )(agg, node_feats, W1, b1.reshape(1, H), W2, b2.reshape(1, D),
      gamma.reshape(1, D), beta.reshape(1, D))


def kernel(node_feats, edge_feats, edge_index, W1, b1, W2, b2, gamma, beta):
    N, D = node_feats.shape
    E = edge_feats.shape[0]
    DC = D // 2

    src1 = edge_index[0]
    dst1 = edge_index[1]
    node_lo = node_feats[:, :DC]
    node_hi = node_feats[:, DC:]

    sc_seg_sum = _make_sc_segment_sum(N, E, D)
    agg = sc_seg_sum(node_lo, node_hi, edge_feats, src1, dst1)

    return _tc_mlp_ln(agg, node_feats, W1, b1, W2, b2, gamma, beta)
